# baseline (speedup 1.0000x reference)
import functools

import jax
import jax.numpy as jnp
from jax import lax
from jax.experimental import pallas as pl
from jax.experimental.pallas import tpu as pltpu
from jax.experimental.pallas import tpu_sc as plsc

B = 16384
D = 500
V = 100000
H1 = 64
NC = 2
NS = 16
NW = NC * NS
KMAX = 16
DPAD = NW * KMAX
VQA = 25088
QOFF = (0, VQA, 2 * VQA, 3 * VQA)
QLEN = (VQA, VQA, VQA, 24704)
VTAIL = V - QOFF[3] - QLEN[3]
TLO = V - VTAIL
NBKT = 5
WINV = 2048
NWIN = B // WINV
NWIN_MAX = NWIN + 1
BP = WINV * NWIN_MAX
ORB = B + 16


@functools.cache
def _make_sc_gather():
    mesh = plsc.VectorSubcoreMesh(core_axis_name="c", subcore_axis_name="s")

    @functools.partial(
        pl.kernel,
        mesh=mesh,
        out_type=(
            jax.ShapeDtypeStruct((DPAD, B), jnp.float32),
            jax.ShapeDtypeStruct((DPAD, B), jnp.float32),
            jax.ShapeDtypeStruct((2 * NBKT * BP,), jnp.int32),
            jax.ShapeDtypeStruct((2 * NBKT * BP,), jnp.int32),
            jax.ShapeDtypeStruct((2 * NBKT * 128,), jnp.int32),
        ),
        scratch_types=[
            pltpu.VMEM((1, VQA), jnp.float32),
            pltpu.VMEM((1, VQA), jnp.float32),
            pltpu.VMEM((VTAIL,), jnp.float32),
            pltpu.VMEM((16,), jnp.int32),
            pltpu.VMEM((16,), jnp.int32),
            pltpu.VMEM((WINV,), jnp.int32),
            pltpu.VMEM((WINV,), jnp.int32),
            pltpu.VMEM((WINV,), jnp.int32),
            pltpu.VMEM((BP,), jnp.int32),
            pltpu.VMEM((BP,), jnp.int32),
            pltpu.VMEM((NBKT * 128,), jnp.int32),
            pltpu.VMEM((ORB,), jnp.float32),
            pltpu.SemaphoreType.DMA,
            pltpu.SemaphoreType.DMA,
            pltpu.SemaphoreType.DMA,
            pltpu.SemaphoreType.DMA,
        ],
        compiler_params=pltpu.CompilerParams(needs_layout_passes=False),
    )
    def _sc_gather(uid_hbm, iid_hbm, utabT_hbm, itabT_hbm, utail_hbm,
                   itail_hbm, uoutT_hbm, ioutT_hbm, bidx_hbm, bpos_hbm,
                   cnts_hbm, slabA_v, slabB_v, tail_v, dscA_v, dscB_v,
                   widx_v, wpos_v, scan_v, bldi_v, bldp_v, cnt_v,
                   outrow_v, ssem, wsem, osem, tsem):
        core = lax.axis_index("c")
        sub = lax.axis_index("s")
        wid = sub * NC + core
        lanes = lax.iota(jnp.int32, 16)
        slabs = (slabA_v, slabB_v)
        dscs = (dscA_v, dscB_v)

        @pl.when(sub < NBKT)
        def _bucket():
            bkt = sub
            lo = jnp.where(bkt == 4, TLO, bkt * VQA)
            hi = jnp.where(bkt >= 3, jnp.where(bkt == 4, V, TLO), lo + VQA)
            for tbl, idx_hbm in ((0, uid_hbm), (1, iid_hbm)):
                def scan_win(w, cnt):
                    pltpu.sync_copy(idx_hbm.at[pl.ds(w * WINV, WINV)], scan_v)

                    def scan_vec(v, cnt):
                        iv = scan_v[pl.ds(v * 16, 16)]
                        m = (iv >= lo) & (iv < hi)
                        plsc.store_compressed(bldi_v.at[pl.ds(cnt, 16)],
                                              iv - lo, mask=m)
                        pv = lanes + (w * WINV + v * 16)
                        plsc.store_compressed(bldp_v.at[pl.ds(cnt, 16)],
                                              pv, mask=m)
                        return cnt + plsc.all_reduce_population_count(m)[0]

                    return lax.fori_loop(0, WINV // 16, scan_vec, cnt)

                cnt = lax.fori_loop(0, NWIN, scan_win, jnp.int32(0))
                pad = lax.rem(16 - lax.rem(cnt, 16), 16)
                plsc.store_compressed(bldi_v.at[pl.ds(cnt, 16)],
                                      jnp.zeros((16,), jnp.int32),
                                      mask=lanes < pad)
                plsc.store_compressed(bldp_v.at[pl.ds(cnt, 16)],
                                      jnp.full((16,), B, jnp.int32) + lanes,
                                      mask=lanes < pad)
                nvec = (cnt + pad) // 16
                scan_v[pl.ds(0, 16)] = jnp.full((16,), nvec, jnp.int32)
                boff = pl.multiple_of(tbl * NBKT * BP + bkt * BP, 128)
                pltpu.sync_copy(bldi_v, bidx_hbm.at[pl.ds(boff, BP)])
                pltpu.sync_copy(bldp_v, bpos_hbm.at[pl.ds(boff, BP)])
                coff = pl.multiple_of(tbl * NBKT * 128 + bkt * 128, 128)
                pltpu.sync_copy(scan_v.at[pl.ds(0, 16)],
                                cnts_hbm.at[pl.ds(coff, 16)])

        plsc.subcore_barrier()

        for tbl in (0, 1):
            tabT_hbm = (utabT_hbm, itabT_hbm)[tbl]
            tail_hbm = (utail_hbm, itail_hbm)[tbl]
            outT_hbm = (uoutT_hbm, ioutT_hbm)[tbl]
            pltpu.sync_copy(
                cnts_hbm.at[pl.ds(tbl * NBKT * 128, NBKT * 128)], cnt_v)
            d0_eff = jnp.minimum(wid, D - 1)
            dscA_v[pl.ds(0, 16)] = jnp.full((16,), d0_eff, jnp.int32)
            pltpu.async_copy(
                tabT_hbm.at[dscA_v.at[pl.ds(0, 1)], pl.ds(0, QLEN[0])],
                slabA_v.at[:, pl.ds(0, QLEN[0])], ssem)

            def per_k(k, _):
                d = wid + NW * k
                d_eff = jnp.minimum(d, D - 1)
                pltpu.async_copy(tail_hbm.at[d_eff], tail_v, tsem)

                @pl.when(k > 0)
                def _():
                    pltpu.make_async_copy(
                        outrow_v.at[pl.ds(0, B)], outT_hbm.at[0], osem).wait()

                def win_loop(q, src_ref, nvec, twod):
                    zv = lanes * 0
                    nwin = (nvec + 127) // 128

                    def one_win(w, _):
                        woff = pl.multiple_of(
                            tbl * NBKT * BP + q * BP + w * WINV, WINV)
                        pltpu.async_copy(
                            bidx_hbm.at[pl.ds(woff, WINV)], widx_v, wsem)
                        pltpu.async_copy(
                            bpos_hbm.at[pl.ds(woff, WINV)], wpos_v, wsem)
                        pltpu.make_async_copy(
                            bidx_hbm.at[pl.ds(0, WINV)], widx_v, wsem).wait()
                        pltpu.make_async_copy(
                            bidx_hbm.at[pl.ds(0, WINV)], wpos_v, wsem).wait()
                        vcnt = jnp.minimum(nvec - w * 128, 128)

                        @plsc.parallel_loop(0, vcnt, unroll=8)
                        def _gv(v):
                            iv = widx_v[pl.ds(v * 16, 16)]
                            pv = wpos_v[pl.ds(v * 16, 16)]
                            if twod:
                                vals = plsc.load_gather(src_ref, [zv, iv])
                            else:
                                vals = plsc.load_gather(src_ref, [iv])
                            plsc.store_scatter(outrow_v, [pv], vals)

                        return 0

                    lax.fori_loop(0, nwin, one_win, 0)

                for q in range(4):
                    if q < 3:
                        dn = d
                        qn = q + 1
                    else:
                        dn = d + NW
                        qn = 0
                    dn_eff = jnp.minimum(dn, D - 1)
                    nbuf = dscs[(q + 1) % 2]
                    nbuf[pl.ds(0, 16)] = jnp.full((16,), dn_eff, jnp.int32)
                    pltpu.async_copy(
                        tabT_hbm.at[nbuf.at[pl.ds(0, 1)],
                                    pl.ds(QOFF[qn], QLEN[qn])],
                        slabs[(q + 1) % 2].at[:, pl.ds(0, QLEN[qn])], ssem)
                    pltpu.make_async_copy(
                        tabT_hbm.at[dscs[q % 2].at[pl.ds(0, 1)],
                                    pl.ds(QOFF[q], QLEN[q])],
                        slabs[q % 2].at[:, pl.ds(0, QLEN[q])], ssem).wait()
                    win_loop(q, slabs[q % 2], cnt_v[pl.ds(q * 128, 16)][0],
                             True)

                pltpu.make_async_copy(tail_hbm.at[0], tail_v, tsem).wait()
                win_loop(4, tail_v, cnt_v[pl.ds(4 * 128, 16)][0], False)
                pltpu.async_copy(outrow_v.at[pl.ds(0, B)], outT_hbm.at[d],
                                 osem)
                return 0

            lax.fori_loop(0, KMAX, per_k, 0)
            pltpu.make_async_copy(
                outrow_v.at[pl.ds(0, B)], outT_hbm.at[0], osem).wait()
            pltpu.make_async_copy(
                tabT_hbm.at[dscA_v.at[pl.ds(0, 1)], pl.ds(0, QLEN[0])],
                slabA_v.at[:, pl.ds(0, QLEN[0])], ssem).wait()

    return _sc_gather


def _tc_dense_body(u_ref, i_ref, wu_ref, wi_ref, bu_ref, bi_ref, wp_ref,
                   bp_ref, o_ref):
    dn = (((0,), (0,)), ((), ()))
    u = lax.dot_general(u_ref[...], wu_ref[...], dn,
                        preferred_element_type=jnp.float32)
    u = jnp.maximum(u + bu_ref[...], 0.0)
    i = lax.dot_general(i_ref[...], wi_ref[...], dn,
                        preferred_element_type=jnp.float32)
    i = jnp.maximum(i + bi_ref[...], 0.0)
    r = jnp.dot(u, wp_ref[:H1, :], preferred_element_type=jnp.float32)
    r = r + jnp.dot(i, wp_ref[H1:, :], preferred_element_type=jnp.float32)
    o_ref[...] = r + bp_ref[...]


BB = 2048


def _tc_dense(uT_docs, iT_docs, wu, wi, bu, bi, wp, bp):
    grid = (B // BB,)
    return pl.pallas_call(
        _tc_dense_body,
        grid=grid,
        in_specs=[
            pl.BlockSpec((DPAD, BB), lambda b: (0, b)),
            pl.BlockSpec((DPAD, BB), lambda b: (0, b)),
            pl.BlockSpec((DPAD, H1), lambda b: (0, 0)),
            pl.BlockSpec((DPAD, H1), lambda b: (0, 0)),
            pl.BlockSpec((1, H1), lambda b: (0, 0)),
            pl.BlockSpec((1, H1), lambda b: (0, 0)),
            pl.BlockSpec((2 * H1, 1), lambda b: (0, 0)),
            pl.BlockSpec((1, 1), lambda b: (0, 0)),
        ],
        out_specs=pl.BlockSpec((BB, 1), lambda b: (b, 0)),
        out_shape=jax.ShapeDtypeStruct((B, 1), jnp.float32),
    )(uT_docs, iT_docs, wu, wi, bu, bi, wp, bp)


def kernel(batch_uid, batch_iid, uid_userDoc, iid_itemDoc, userFC_W, userFC_b,
           itemFC_W, itemFC_b, pred_W, pred_b):
    uid = batch_uid.astype(jnp.int32)
    iid = batch_iid.astype(jnp.int32)
    uT = uid_userDoc.T
    iT = iid_itemDoc.T
    utail = lax.slice(uT, (0, TLO), (D, V))
    itail = lax.slice(iT, (0, TLO), (D, V))
    uT_docs, iT_docs, _, _, _ = _make_sc_gather()(uid, iid, uT, iT,
                                                  utail, itail)
    wu_pad = jnp.pad(userFC_W, ((0, DPAD - D), (0, 0)))
    wi_pad = jnp.pad(itemFC_W, ((0, DPAD - D), (0, 0)))
    out = _tc_dense(uT_docs, iT_docs, wu_pad, wi_pad,
                    userFC_b.reshape(1, H1), itemFC_b.reshape(1, H1),
                    pred_W, pred_b.reshape(1, 1))
    return out


# speedup vs baseline: 1.7624x; 1.7624x over previous
import functools

import jax
import jax.numpy as jnp
from jax import lax
from jax.experimental import pallas as pl
from jax.experimental.pallas import tpu as pltpu
from jax.experimental.pallas import tpu_sc as plsc

B = 16384
D = 500
V = 100000
H1 = 64
NC = 2
NS = 16
NW = NC * NS
KMAX = 16
DPAD = NW * KMAX
VQA = 25088
QOFF = (0, VQA, 2 * VQA, 3 * VQA)
QLEN = (VQA, VQA, VQA, 24704)
TLO = QOFF[3] + QLEN[3]
VTAIL = V - TLO
NBKT = 5
BLO = (0, VQA, 2 * VQA, 3 * VQA, TLO)
BHI = (VQA, 2 * VQA, 3 * VQA, TLO, V)
WINV = 2048
NWIN = B // WINV
BRES = B + 16 * NBKT
ORB = B + 16


@functools.cache
def _make_sc_gather():
    mesh = plsc.VectorSubcoreMesh(core_axis_name="c", subcore_axis_name="s")

    @functools.partial(
        pl.kernel,
        mesh=mesh,
        out_type=(
            jax.ShapeDtypeStruct((DPAD, B), jnp.float32),
            jax.ShapeDtypeStruct((DPAD, B), jnp.float32),
        ),
        scratch_types=[
            pltpu.VMEM((1, VQA), jnp.float32),
            pltpu.VMEM((1, VQA), jnp.float32),
            pltpu.VMEM((VTAIL,), jnp.float32),
            pltpu.VMEM((16,), jnp.int32),
            pltpu.VMEM((16,), jnp.int32),
            pltpu.VMEM((WINV,), jnp.int32),
            pltpu.VMEM((BRES,), jnp.int32),
            pltpu.VMEM((BRES,), jnp.int32),
            pltpu.VMEM((ORB,), jnp.float32),
            pltpu.SemaphoreType.DMA,
            pltpu.SemaphoreType.DMA,
            pltpu.SemaphoreType.DMA,
        ],
        compiler_params=pltpu.CompilerParams(needs_layout_passes=False),
    )
    def _sc_gather(uid_hbm, iid_hbm, utabT_hbm, itabT_hbm, utail_hbm,
                   itail_hbm, uoutT_hbm, ioutT_hbm, slabA_v, slabB_v,
                   tail_v, dscA_v, dscB_v, scan_v, bidx_v, bpos_v,
                   outrow_v, ssem, osem, tsem):
        core = lax.axis_index("c")
        sub = lax.axis_index("s")
        wid = sub * NC + core
        lanes = lax.iota(jnp.int32, 16)
        slabs = (slabA_v, slabB_v)
        dscs = (dscA_v, dscB_v)

        for tbl in (0, 1):
            tabT_hbm = (utabT_hbm, itabT_hbm)[tbl]
            tail_hbm = (utail_hbm, itail_hbm)[tbl]
            outT_hbm = (uoutT_hbm, ioutT_hbm)[tbl]
            idx_hbm = (uid_hbm, iid_hbm)[tbl]

            def count_win(w, cnts):
                pltpu.sync_copy(idx_hbm.at[pl.ds(w * WINV, WINV)], scan_v)

                def count_vec(v, cnts):
                    iv = scan_v[pl.ds(v * 16, 16)]
                    out = []
                    for b in range(NBKT):
                        m = (iv >= BLO[b]) & (iv < BHI[b])
                        out.append(
                            cnts[b] + plsc.all_reduce_population_count(m)[0])
                    return tuple(out)

                return lax.fori_loop(0, WINV // 16, count_vec, cnts)

            cnts = lax.fori_loop(0, NWIN, count_win,
                                 tuple(jnp.int32(0) for _ in range(NBKT)))
            nvecs = tuple((c + 15) // 16 for c in cnts)
            starts = []
            acc = jnp.int32(0)
            for b in range(NBKT):
                starts.append(acc)
                acc = acc + nvecs[b] * 16
            starts = tuple(starts)

            def fill_win(w, curs):
                pltpu.sync_copy(idx_hbm.at[pl.ds(w * WINV, WINV)], scan_v)

                def fill_vec(v, curs):
                    iv = scan_v[pl.ds(v * 16, 16)]
                    pv = lanes + (w * WINV + v * 16)
                    out = []
                    for b in range(NBKT):
                        m = (iv >= BLO[b]) & (iv < BHI[b])
                        plsc.store_compressed(bidx_v.at[pl.ds(curs[b], 16)],
                                              iv - BLO[b], mask=m)
                        plsc.store_compressed(bpos_v.at[pl.ds(curs[b], 16)],
                                              pv, mask=m)
                        out.append(
                            curs[b] + plsc.all_reduce_population_count(m)[0])
                    return tuple(out)

                return lax.fori_loop(0, WINV // 16, fill_vec, curs)

            curs = lax.fori_loop(0, NWIN, fill_win, starts)
            for b in range(NBKT):
                pad = starts[b] + nvecs[b] * 16 - curs[b]
                plsc.store_compressed(bidx_v.at[pl.ds(curs[b], 16)],
                                      jnp.zeros((16,), jnp.int32),
                                      mask=lanes < pad)
                plsc.store_compressed(bpos_v.at[pl.ds(curs[b], 16)],
                                      jnp.full((16,), B, jnp.int32) + lanes,
                                      mask=lanes < pad)

            d0_eff = jnp.minimum(wid, D - 1)
            dscA_v[pl.ds(0, 16)] = jnp.full((16,), d0_eff, jnp.int32)
            pltpu.async_copy(
                tabT_hbm.at[dscA_v.at[pl.ds(0, 1)], pl.ds(0, QLEN[0])],
                slabA_v.at[:, pl.ds(0, QLEN[0])], ssem)

            def per_k(k, _):
                d = wid + NW * k
                d_eff = jnp.minimum(d, D - 1)
                pltpu.async_copy(tail_hbm.at[d_eff], tail_v, tsem)

                @pl.when(k > 0)
                def _():
                    pltpu.make_async_copy(
                        outrow_v.at[pl.ds(0, B)], outT_hbm.at[0], osem).wait()

                def bucket_gather(b, src_ref, twod):
                    lo16 = starts[b] // 16
                    zv = lanes * 0

                    @plsc.parallel_loop(lo16, lo16 + nvecs[b], unroll=8)
                    def _gv(v):
                        iv = bidx_v[pl.ds(v * 16, 16)]
                        pv = bpos_v[pl.ds(v * 16, 16)]
                        if twod:
                            vals = plsc.load_gather(src_ref, [zv, iv])
                        else:
                            vals = plsc.load_gather(src_ref, [iv])
                        plsc.store_scatter(outrow_v, [pv], vals)

                for q in range(4):
                    if q < 3:
                        dn = d
                        qn = q + 1
                    else:
                        dn = d + NW
                        qn = 0
                    dn_eff = jnp.minimum(dn, D - 1)
                    nbuf = dscs[(q + 1) % 2]
                    nbuf[pl.ds(0, 16)] = jnp.full((16,), dn_eff, jnp.int32)
                    pltpu.async_copy(
                        tabT_hbm.at[nbuf.at[pl.ds(0, 1)],
                                    pl.ds(QOFF[qn], QLEN[qn])],
                        slabs[(q + 1) % 2].at[:, pl.ds(0, QLEN[qn])], ssem)
                    pltpu.make_async_copy(
                        tabT_hbm.at[dscs[q % 2].at[pl.ds(0, 1)],
                                    pl.ds(QOFF[q], QLEN[q])],
                        slabs[q % 2].at[:, pl.ds(0, QLEN[q])], ssem).wait()
                    bucket_gather(q, slabs[q % 2], True)

                pltpu.make_async_copy(tail_hbm.at[0], tail_v, tsem).wait()
                bucket_gather(4, tail_v, False)
                pltpu.async_copy(outrow_v.at[pl.ds(0, B)], outT_hbm.at[d],
                                 osem)
                return 0

            lax.fori_loop(0, KMAX, per_k, 0)
            pltpu.make_async_copy(
                outrow_v.at[pl.ds(0, B)], outT_hbm.at[0], osem).wait()
            pltpu.make_async_copy(
                tabT_hbm.at[dscA_v.at[pl.ds(0, 1)], pl.ds(0, QLEN[0])],
                slabA_v.at[:, pl.ds(0, QLEN[0])], ssem).wait()

    return _sc_gather


def _tc_dense_body(u_ref, i_ref, wu_ref, wi_ref, bu_ref, bi_ref, wp_ref,
                   bp_ref, o_ref):
    dn = (((0,), (0,)), ((), ()))
    u = lax.dot_general(u_ref[...], wu_ref[...], dn,
                        preferred_element_type=jnp.float32)
    u = jnp.maximum(u + bu_ref[...], 0.0)
    i = lax.dot_general(i_ref[...], wi_ref[...], dn,
                        preferred_element_type=jnp.float32)
    i = jnp.maximum(i + bi_ref[...], 0.0)
    r = jnp.dot(u, wp_ref[:H1, :], preferred_element_type=jnp.float32)
    r = r + jnp.dot(i, wp_ref[H1:, :], preferred_element_type=jnp.float32)
    o_ref[...] = r + bp_ref[...]


BB = 2048


def _tc_dense(uT_docs, iT_docs, wu, wi, bu, bi, wp, bp):
    grid = (B // BB,)
    return pl.pallas_call(
        _tc_dense_body,
        grid=grid,
        in_specs=[
            pl.BlockSpec((DPAD, BB), lambda b: (0, b)),
            pl.BlockSpec((DPAD, BB), lambda b: (0, b)),
            pl.BlockSpec((DPAD, H1), lambda b: (0, 0)),
            pl.BlockSpec((DPAD, H1), lambda b: (0, 0)),
            pl.BlockSpec((1, H1), lambda b: (0, 0)),
            pl.BlockSpec((1, H1), lambda b: (0, 0)),
            pl.BlockSpec((2 * H1, 1), lambda b: (0, 0)),
            pl.BlockSpec((1, 1), lambda b: (0, 0)),
        ],
        out_specs=pl.BlockSpec((BB, 1), lambda b: (b, 0)),
        out_shape=jax.ShapeDtypeStruct((B, 1), jnp.float32),
    )(uT_docs, iT_docs, wu, wi, bu, bi, wp, bp)


def kernel(batch_uid, batch_iid, uid_userDoc, iid_itemDoc, userFC_W, userFC_b,
           itemFC_W, itemFC_b, pred_W, pred_b):
    uid = batch_uid.astype(jnp.int32)
    iid = batch_iid.astype(jnp.int32)
    uT = uid_userDoc.T
    iT = iid_itemDoc.T
    utail = lax.slice(uT, (0, TLO), (D, V))
    itail = lax.slice(iT, (0, TLO), (D, V))
    uT_docs, iT_docs = _make_sc_gather()(uid, iid, uT, iT, utail, itail)
    wu_pad = jnp.pad(userFC_W, ((0, DPAD - D), (0, 0)))
    wi_pad = jnp.pad(itemFC_W, ((0, DPAD - D), (0, 0)))
    out = _tc_dense(uT_docs, iT_docs, wu_pad, wi_pad,
                    userFC_b.reshape(1, H1), itemFC_b.reshape(1, H1),
                    pred_W, pred_b.reshape(1, 1))
    return out


# speedup vs baseline: 1.9080x; 1.0826x over previous
import functools

import jax
import jax.numpy as jnp
from jax import lax
from jax.experimental import pallas as pl
from jax.experimental.pallas import tpu as pltpu
from jax.experimental.pallas import tpu_sc as plsc

B = 16384
D = 500
V = 100000
H1 = 64
NC = 2
NS = 16
NW = NC * NS
KMAX = (D + NW - 1) // NW
OC = 4096
NOC = B // OC


@functools.cache
def _make_sc_gather():
    mesh = plsc.VectorSubcoreMesh(core_axis_name="c", subcore_axis_name="s")

    @functools.partial(
        pl.kernel,
        mesh=mesh,
        out_type=(
            jax.ShapeDtypeStruct((D, B), jnp.float32),
            jax.ShapeDtypeStruct((D, B), jnp.float32),
        ),
        scratch_types=[
            pltpu.VMEM((B,), jnp.int32),
            pltpu.VMEM((V,), jnp.float32),
            pltpu.VMEM((2, OC), jnp.float32),
            pltpu.SemaphoreType.DMA,
            pltpu.SemaphoreType.DMA,
            pltpu.SemaphoreType.DMA,
        ],
        compiler_params=pltpu.CompilerParams(needs_layout_passes=False),
    )
    def _sc_gather(uid_hbm, iid_hbm, utabT_hbm, itabT_hbm, uoutT_hbm,
                   ioutT_hbm, idx_v, row_v, out_v, rsem, osem, xsem):
        wid = lax.axis_index("s") * NC + lax.axis_index("c")

        def one_table(idx_hbm, tabT_hbm, outT_hbm):
            pltpu.sync_copy(idx_hbm, idx_v)

            def per_d(k, _):
                d = wid + NW * k

                @pl.when(d < D)
                def _():
                    pltpu.async_copy(tabT_hbm.at[d], row_v, rsem).wait()
                    for c in range(NOC):
                        buf = c % 2
                        if c >= 2:
                            pltpu.make_async_copy(
                                out_v.at[buf], outT_hbm.at[0, pl.ds(0, OC)],
                                osem).wait()

                        @plsc.parallel_loop(0, OC // 16, unroll=32)
                        def _gather16(v):
                            iv = idx_v[pl.ds(c * OC + v * 16, 16)]
                            out_v[buf, pl.ds(v * 16, 16)] = plsc.load_gather(
                                row_v, [iv])
                        pltpu.async_copy(
                            out_v.at[buf], outT_hbm.at[d, pl.ds(c * OC, OC)],
                            osem)
                    pltpu.make_async_copy(
                        out_v.at[0], outT_hbm.at[0, pl.ds(0, OC)], osem).wait()
                    pltpu.make_async_copy(
                        out_v.at[1], outT_hbm.at[0, pl.ds(0, OC)], osem).wait()

                return 0

            lax.fori_loop(0, KMAX, per_d, 0)

        one_table(uid_hbm, utabT_hbm, uoutT_hbm)
        one_table(iid_hbm, itabT_hbm, ioutT_hbm)

    return _sc_gather


def _tc_dense_body(u_ref, i_ref, wu_ref, wi_ref, bu_ref, bi_ref, wp_ref,
                   bp_ref, o_ref):
    dn = (((0,), (0,)), ((), ()))
    u = lax.dot_general(u_ref[...], wu_ref[...], dn,
                        preferred_element_type=jnp.float32)
    u = jnp.maximum(u + bu_ref[...], 0.0)
    i = lax.dot_general(i_ref[...], wi_ref[...], dn,
                        preferred_element_type=jnp.float32)
    i = jnp.maximum(i + bi_ref[...], 0.0)
    r = jnp.dot(u, wp_ref[:H1, :], preferred_element_type=jnp.float32)
    r = r + jnp.dot(i, wp_ref[H1:, :], preferred_element_type=jnp.float32)
    o_ref[...] = r + bp_ref[...]


BB = 2048


def _tc_dense(uT_docs, iT_docs, wu, wi, bu, bi, wp, bp):
    grid = (B // BB,)
    return pl.pallas_call(
        _tc_dense_body,
        grid=grid,
        in_specs=[
            pl.BlockSpec((D, BB), lambda b: (0, b)),
            pl.BlockSpec((D, BB), lambda b: (0, b)),
            pl.BlockSpec((D, H1), lambda b: (0, 0)),
            pl.BlockSpec((D, H1), lambda b: (0, 0)),
            pl.BlockSpec((1, H1), lambda b: (0, 0)),
            pl.BlockSpec((1, H1), lambda b: (0, 0)),
            pl.BlockSpec((2 * H1, 1), lambda b: (0, 0)),
            pl.BlockSpec((1, 1), lambda b: (0, 0)),
        ],
        out_specs=pl.BlockSpec((BB, 1), lambda b: (b, 0)),
        out_shape=jax.ShapeDtypeStruct((B, 1), jnp.float32),
    )(uT_docs, iT_docs, wu, wi, bu, bi, wp, bp)


def kernel(batch_uid, batch_iid, uid_userDoc, iid_itemDoc, userFC_W, userFC_b,
           itemFC_W, itemFC_b, pred_W, pred_b):
    uid = batch_uid.astype(jnp.int32)
    iid = batch_iid.astype(jnp.int32)
    uT_docs, iT_docs = _make_sc_gather()(uid, iid, uid_userDoc.T,
                                         iid_itemDoc.T)
    out = _tc_dense(uT_docs, iT_docs, userFC_W, itemFC_W,
                    userFC_b.reshape(1, H1), itemFC_b.reshape(1, H1),
                    pred_W, pred_b.reshape(1, 1))
    return out
